# baseline (speedup 1.0000x reference)
import math

import numpy as np

import jax
import jax.numpy as jnp
from jax import lax
from jax.experimental import pallas as pl
from jax.experimental.pallas import tpu as pltpu

_LOG2E = 1.4426950408889634
_BIG = 1e30


def _round_up(x, m):
    return ((x + m - 1) // m) * m


def _pick_tile(n):
    n_pad = _round_up(n, 128)
    for t in (2048, 1024, 512, 256):
        if n_pad % t == 0:
            return t
    return 128


def _fused_tile_kernel(ab_t, bb_t, ob_t, fl_t, a_ref, b_ref, qn_ref, o_ref):
    s = pl.program_id(0)
    flag = fl_t[s]

    @pl.when(flag == 1)
    def _init():
        o_ref[...] = jnp.zeros_like(o_ref)

    dots = lax.dot_general(
        a_ref[...], b_ref[...], (((1,), (1,)), ((), ())),
        preferred_element_type=jnp.float32,
    )
    e = jnp.exp2(dots - qn_ref[0])
    rows = jnp.sum(e, axis=-1, keepdims=True)
    w = jnp.where(flag == 2, 2.0, 1.0).astype(jnp.float32)
    o_ref[...] = o_ref[...] + rows * w


def _prep(P, T, ksize):
    n, d = P.shape
    P32 = P.astype(jnp.float32)
    q = _LOG2E / float(ksize)
    n_pad = _round_up(n, T)
    d_pad = _round_up(d, 128)
    if n_pad != n or d_pad != d:
        P32 = jnp.zeros((n_pad, d_pad), jnp.float32).at[:n, :d].set(P32)
    scaled = (P32 * (2.0 * q)).astype(jnp.bfloat16)
    plain = P32.astype(jnp.bfloat16)
    qn = jnp.sum(P32 * P32, axis=-1) * q
    if n_pad != n:
        qn = jnp.where(jnp.arange(n_pad) < n, qn, _BIG)
    rowfac = jnp.exp2(-qn)
    return scaled, plain, qn, rowfac


def _gram_rowsums(scaled, plain, qn_rows, T, steps):
    rows_total, D = scaled.shape
    nblk = rows_total // T
    ab = np.asarray([t[0] for t in steps], np.int32)
    bb = np.asarray([t[1] for t in steps], np.int32)
    ob = np.asarray([t[2] for t in steps], np.int32)
    fl = np.asarray([t[3] for t in steps], np.int32)
    n_out_blocks = int(ob.max()) + 1

    out = pl.pallas_call(
        _fused_tile_kernel,
        out_shape=jax.ShapeDtypeStruct((n_out_blocks * T, 1), jnp.float32),
        grid_spec=pltpu.PrefetchScalarGridSpec(
            num_scalar_prefetch=4,
            grid=(len(steps),),
            in_specs=[
                pl.BlockSpec((T, D), lambda s, a_t, b_t, o_t, f_t: (a_t[s], 0)),
                pl.BlockSpec((T, D), lambda s, a_t, b_t, o_t, f_t: (b_t[s], 0)),
                pl.BlockSpec((1, 1, T), lambda s, a_t, b_t, o_t, f_t: (b_t[s], 0, 0)),
            ],
            out_specs=pl.BlockSpec((T, 1), lambda s, a_t, b_t, o_t, f_t: (o_t[s], 0)),
        ),
        compiler_params=pltpu.CompilerParams(
            dimension_semantics=("arbitrary",),
            vmem_limit_bytes=100 * 1024 * 1024,
        ),
    )(jnp.asarray(ab), jnp.asarray(bb), jnp.asarray(ob), jnp.asarray(fl),
      scaled, plain, qn_rows)
    return out[:, 0]


def _tri_steps(nt, blk_off, out_off):
    steps = []
    for i in range(nt):
        for j in range(i, nt):
            flag = 1 if j == i else 2
            steps.append((blk_off + i, blk_off + j, out_off + i, flag))
    return steps


def _full_steps(nti, ntj, a_off, b_off, out_off):
    steps = []
    for i in range(nti):
        for j in range(ntj):
            steps.append((a_off + i, b_off + j, out_off + i, 1 if j == 0 else 0))
    return steps


def kernel(X, Z):
    ksize = 64.0
    N, D = X.shape
    M, D2 = Z.shape
    assert D == D2, "feature dims must match"
    norm = math.sqrt(2.0 * math.pi * ksize)

    T = min(_pick_tile(N), _pick_tile(M))
    Xs, Xp, qn_x, fac_x = _prep(X, T, ksize)
    Zs, Zp, qn_z, fac_z = _prep(Z, T, ksize)
    ntx, ntz = Xs.shape[0] // T, Zs.shape[0] // T

    scaled = jnp.concatenate([Xs, Zs], axis=0)
    plain = jnp.concatenate([Xp, Zp], axis=0)
    qn_rows = jnp.concatenate([qn_x, qn_z]).reshape(ntx + ntz, 1, T)

    steps = (
        _tri_steps(ntx, 0, 0)
        + _tri_steps(ntz, ntx, ntx)
        + _full_steps(ntx, ntz, 0, ntx, ntx + ntz)
    )
    rs = _gram_rowsums(scaled, plain, qn_rows, T, steps)

    nx, nz = ntx * T, ntz * T
    s_xx = jnp.sum(fac_x * rs[:nx])
    s_zz = jnp.sum(fac_z * rs[nx:nx + nz])
    s_xz = jnp.sum(fac_x * rs[nx + nz:])

    m_xx = s_xx / (norm * N * N)
    m_zz = s_zz / (norm * M * M)
    m_xz = s_xz / (norm * N * M)
    return jnp.log(jnp.sqrt(m_xx * m_zz + 1e-5) / (m_xz + 1e-5))


# speedup vs baseline: 1.0322x; 1.0322x over previous
import math

import numpy as np

import jax
import jax.numpy as jnp
from jax import lax
from jax.experimental import pallas as pl
from jax.experimental.pallas import tpu as pltpu

_LOG2E = 1.4426950408889634
_BIG = 1e30


def _round_up(x, m):
    return ((x + m - 1) // m) * m


def _pick_tile(n):
    n_pad = _round_up(n, 128)
    for t in (2048, 1024, 512, 256):
        if n_pad % t == 0:
            return t
    return 128


def _fused_tile_kernel(ab_t, bb_t, ob_t, fl_t, a_ref, b_ref, qn_ref, o_ref):
    s = pl.program_id(0)
    flag = fl_t[s]

    @pl.when(flag == 1)
    def _init():
        o_ref[...] = jnp.zeros_like(o_ref)

    dots = lax.dot_general(
        a_ref[...], b_ref[...], (((1,), (1,)), ((), ())),
        preferred_element_type=jnp.float32,
    )
    e = jnp.exp2(dots - qn_ref[0])
    rows = jnp.sum(e, axis=-1, keepdims=True)
    w = jnp.where(flag == 2, 2.0, 1.0).astype(jnp.float32)
    o_ref[...] = o_ref[...] + rows * w


def _prep(P, T, ksize):
    n, d = P.shape
    P32 = P.astype(jnp.float32)
    q = _LOG2E / float(ksize)
    n_pad = _round_up(n, T)
    d_pad = _round_up(d, 128)
    if n_pad != n or d_pad != d:
        P32 = jnp.zeros((n_pad, d_pad), jnp.float32).at[:n, :d].set(P32)
    scaled = (P32 * math.sqrt(2.0 * q)).astype(jnp.bfloat16)
    qn = jnp.sum(P32 * P32, axis=-1) * q
    if n_pad != n:
        qn = jnp.where(jnp.arange(n_pad) < n, qn, _BIG)
    rowfac = jnp.exp2(-qn)
    return scaled, qn, rowfac


def _gram_rowsums(data, qn_rows, T, steps):
    rows_total, D = data.shape
    nblk = rows_total // T
    ab = np.asarray([t[0] for t in steps], np.int32)
    bb = np.asarray([t[1] for t in steps], np.int32)
    ob = np.asarray([t[2] for t in steps], np.int32)
    fl = np.asarray([t[3] for t in steps], np.int32)
    n_out_blocks = int(ob.max()) + 1

    out = pl.pallas_call(
        _fused_tile_kernel,
        out_shape=jax.ShapeDtypeStruct((n_out_blocks * T, 128), jnp.float32),
        grid_spec=pltpu.PrefetchScalarGridSpec(
            num_scalar_prefetch=4,
            grid=(len(steps),),
            in_specs=[
                pl.BlockSpec((T, D), lambda s, a_t, b_t, o_t, f_t: (a_t[s], 0)),
                pl.BlockSpec((T, D), lambda s, a_t, b_t, o_t, f_t: (b_t[s], 0)),
                pl.BlockSpec((1, 1, T), lambda s, a_t, b_t, o_t, f_t: (b_t[s], 0, 0)),
            ],
            out_specs=pl.BlockSpec((T, 128), lambda s, a_t, b_t, o_t, f_t: (o_t[s], 0)),
        ),
        compiler_params=pltpu.CompilerParams(
            dimension_semantics=("arbitrary",),
            vmem_limit_bytes=100 * 1024 * 1024,
        ),
    )(jnp.asarray(ab), jnp.asarray(bb), jnp.asarray(ob), jnp.asarray(fl),
      data, data, qn_rows)
    return out[:, 0]


def _tri_steps(nt, blk_off, out_off):
    steps = []
    for i in range(nt):
        for j in range(i, nt):
            flag = 1 if j == i else 2
            steps.append((blk_off + i, blk_off + j, out_off + i, flag))
    return steps


def _full_steps(nti, ntj, a_off, b_off, out_off):
    steps = []
    for i in range(nti):
        for j in range(ntj):
            steps.append((a_off + i, b_off + j, out_off + i, 1 if j == 0 else 0))
    return steps


def kernel(X, Z):
    ksize = 64.0
    N, D = X.shape
    M, D2 = Z.shape
    assert D == D2, "feature dims must match"
    norm = math.sqrt(2.0 * math.pi * ksize)

    T = min(_pick_tile(N), _pick_tile(M))
    Xc, qn_x, fac_x = _prep(X, T, ksize)
    Zc, qn_z, fac_z = _prep(Z, T, ksize)
    ntx, ntz = Xc.shape[0] // T, Zc.shape[0] // T

    data = jnp.concatenate([Xc, Zc], axis=0)
    qn_rows = jnp.concatenate([qn_x, qn_z]).reshape(ntx + ntz, 1, T)

    steps = (
        _tri_steps(ntx, 0, 0)
        + _tri_steps(ntz, ntx, ntx)
        + _full_steps(ntx, ntz, 0, ntx, ntx + ntz)
    )
    rs = _gram_rowsums(data, qn_rows, T, steps)

    nx, nz = ntx * T, ntz * T
    s_xx = jnp.sum(fac_x * rs[:nx])
    s_zz = jnp.sum(fac_z * rs[nx:nx + nz])
    s_xz = jnp.sum(fac_x * rs[nx + nz:])

    m_xx = s_xx / (norm * N * N)
    m_zz = s_zz / (norm * M * M)
    m_xz = s_xz / (norm * N * M)
    return jnp.log(jnp.sqrt(m_xx * m_zz + 1e-5) / (m_xz + 1e-5))


# speedup vs baseline: 1.1179x; 1.0830x over previous
import math

import numpy as np

import jax
import jax.numpy as jnp
from jax import lax
from jax.experimental import pallas as pl
from jax.experimental.pallas import tpu as pltpu

_LOG2E = 1.4426950408889634
_BIG = 1e30


def _round_up(x, m):
    return ((x + m - 1) // m) * m


def _pick_tile(n):
    n_pad = _round_up(n, 128)
    for t in (2048, 1024, 512, 256):
        if n_pad % t == 0:
            return t
    return 128


def _sym_tile_kernel(ii_ref, jj_ref, a_ref, b_ref, qn_ref, o_ref):
    s0 = pl.program_id(0)
    s1 = pl.program_id(1)
    i = ii_ref[s0, s1]
    j = jj_ref[s0, s1]

    @pl.when(j == i)
    def _init():
        o_ref[...] = jnp.zeros_like(o_ref)

    dots = lax.dot_general(
        a_ref[...], b_ref[...], (((1,), (1,)), ((), ())),
        preferred_element_type=jnp.float32,
    )
    e = jnp.exp2(dots - qn_ref[0])
    rows = jnp.sum(e, axis=-1, keepdims=True)
    w = jnp.where(j > i, 2.0, 1.0).astype(jnp.float32)
    o_ref[...] = o_ref[...] + rows * w


def _cross_tile_kernel(a_ref, b_ref, qn_ref, o_ref):
    j = pl.program_id(1)

    @pl.when(j == 0)
    def _init():
        o_ref[...] = jnp.zeros_like(o_ref)

    dots = lax.dot_general(
        a_ref[...], b_ref[...], (((1,), (1,)), ((), ())),
        preferred_element_type=jnp.float32,
    )
    e = jnp.exp2(dots - qn_ref[0])
    o_ref[...] = o_ref[...] + jnp.sum(e, axis=-1, keepdims=True)


def _sym_rowsums(data, qn, T, nt):
    n_pad, D = data.shape
    if nt % 2 == 0 and nt > 1:
        g0, g1 = nt // 2, nt + 1
        ii = np.zeros((g0, g1), np.int32)
        jj = np.zeros((g0, g1), np.int32)
        for s0 in range(g0):
            r0, r1 = s0, nt - 1 - s0
            tiles = [(r0, j) for j in range(r0, nt)]
            tiles += [(r1, j) for j in range(r1, nt)]
            for s1, (ti, tj) in enumerate(tiles):
                ii[s0, s1], jj[s0, s1] = ti, tj
    else:
        tri = [(i, j) for i in range(nt) for j in range(i, nt)]
        g0, g1 = 1, len(tri)
        ii = np.asarray([t[0] for t in tri], np.int32).reshape(1, -1)
        jj = np.asarray([t[1] for t in tri], np.int32).reshape(1, -1)

    out = pl.pallas_call(
        _sym_tile_kernel,
        out_shape=jax.ShapeDtypeStruct((n_pad, 128), jnp.float32),
        grid_spec=pltpu.PrefetchScalarGridSpec(
            num_scalar_prefetch=2,
            grid=(g0, g1),
            in_specs=[
                pl.BlockSpec((T, D), lambda s0, s1, ii, jj: (ii[s0, s1], 0)),
                pl.BlockSpec((T, D), lambda s0, s1, ii, jj: (jj[s0, s1], 0)),
                pl.BlockSpec((1, 1, T), lambda s0, s1, ii, jj: (jj[s0, s1], 0, 0)),
            ],
            out_specs=pl.BlockSpec((T, 128), lambda s0, s1, ii, jj: (ii[s0, s1], 0)),
        ),
        compiler_params=pltpu.CompilerParams(
            dimension_semantics=("arbitrary", "arbitrary"),
            vmem_limit_bytes=100 * 1024 * 1024,
        ),
    )(jnp.asarray(ii), jnp.asarray(jj), data, data, qn)
    return out[:, 0]


def _cross_rowsums(a_data, b_data, qn_b, TM, TN):
    n_pad, D = a_data.shape
    m_pad, _ = b_data.shape
    out = pl.pallas_call(
        _cross_tile_kernel,
        out_shape=jax.ShapeDtypeStruct((n_pad, 128), jnp.float32),
        grid=(n_pad // TM, m_pad // TN),
        in_specs=[
            pl.BlockSpec((TM, D), lambda i, j: (i, 0)),
            pl.BlockSpec((TN, D), lambda i, j: (j, 0)),
            pl.BlockSpec((1, 1, TN), lambda i, j: (j, 0, 0)),
        ],
        out_specs=pl.BlockSpec((TM, 128), lambda i, j: (i, 0)),
        compiler_params=pltpu.CompilerParams(
            dimension_semantics=("arbitrary", "arbitrary"),
            vmem_limit_bytes=100 * 1024 * 1024,
        ),
    )(a_data, b_data, qn_b)
    return out[:, 0]


def _prep(P, T, ksize):
    n, d = P.shape
    P32 = P.astype(jnp.float32)
    q = _LOG2E / float(ksize)
    n_pad = _round_up(n, T)
    d_pad = _round_up(d, 128)
    if n_pad != n or d_pad != d:
        P32 = jnp.zeros((n_pad, d_pad), jnp.float32).at[:n, :d].set(P32)
    scaled = (P32 * math.sqrt(2.0 * q)).astype(jnp.bfloat16)
    qn = jnp.sum(P32 * P32, axis=-1) * q
    if n_pad != n:
        qn = jnp.where(jnp.arange(n_pad) < n, qn, _BIG)
    rowfac = jnp.exp2(-qn)
    return scaled, qn, rowfac


def kernel(X, Z):
    ksize = 64.0
    N, D = X.shape
    M, D2 = Z.shape
    assert D == D2, "feature dims must match"
    norm = math.sqrt(2.0 * math.pi * ksize)

    Tx = _pick_tile(N)
    Tz = _pick_tile(M)
    Xc, qn_x, fac_x = _prep(X, Tx, ksize)
    Zc, qn_z, fac_z = _prep(Z, Tz, ksize)
    ntx, ntz = Xc.shape[0] // Tx, Zc.shape[0] // Tz
    qx_rows = qn_x.reshape(ntx, 1, Tx)
    qz_rows = qn_z.reshape(ntz, 1, Tz)

    rs_xx = _sym_rowsums(Xc, qx_rows, Tx, ntx)
    rs_zz = _sym_rowsums(Zc, qz_rows, Tz, ntz)
    rs_xz = _cross_rowsums(Xc, Zc, qz_rows, Tx, Tz)

    s_xx = jnp.sum(fac_x * rs_xx)
    s_zz = jnp.sum(fac_z * rs_zz)
    s_xz = jnp.sum(fac_x * rs_xz)

    m_xx = s_xx / (norm * N * N)
    m_zz = s_zz / (norm * M * M)
    m_xz = s_xz / (norm * N * M)
    return jnp.log(jnp.sqrt(m_xx * m_zz + 1e-5) / (m_xz + 1e-5))


# speedup vs baseline: 1.1432x; 1.0226x over previous
import math

import numpy as np

import jax
import jax.numpy as jnp
from jax import lax
from jax.experimental import pallas as pl
from jax.experimental.pallas import tpu as pltpu

_LOG2E = 1.4426950408889634
_BIG = 1e30


def _round_up(x, m):
    return ((x + m - 1) // m) * m


def _pick_tile(n):
    n_pad = _round_up(n, 128)
    for t in (2048, 1024, 512, 256):
        if n_pad % t == 0:
            return t
    return 128


def _sym_tile_kernel(ii_ref, jj_ref, a_ref, b_ref, qn_ref, o_ref):
    s0 = pl.program_id(0)
    s1 = pl.program_id(1)
    i = ii_ref[s0, s1]
    j = jj_ref[s0, s1]

    @pl.when(j == i)
    def _init():
        o_ref[...] = jnp.zeros_like(o_ref)

    dots = lax.dot_general(
        a_ref[...], b_ref[...], (((1,), (1,)), ((), ())),
        preferred_element_type=jnp.float32,
    )
    e = jnp.exp2(dots - qn_ref[0])
    rows = jnp.sum(e, axis=-1, keepdims=True)
    w = jnp.where(j > i, 2.0, 1.0).astype(jnp.float32)
    o_ref[...] = o_ref[...] + rows * w


def _cross_tile_kernel(a_ref, b_ref, qn_ref, o_ref):
    j = pl.program_id(1)

    @pl.when(j == 0)
    def _init():
        o_ref[...] = jnp.zeros_like(o_ref)

    dots = lax.dot_general(
        a_ref[...], b_ref[...], (((1,), (1,)), ((), ())),
        preferred_element_type=jnp.float32,
    )
    e = jnp.exp2(dots - qn_ref[0])
    o_ref[...] = o_ref[...] + jnp.sum(e, axis=-1, keepdims=True)


def _sym_rowsums(data, qn, T, nt):
    n_pad, D = data.shape
    if nt % 2 == 0 and nt > 1:
        g0, g1 = nt // 2, nt + 1
        ii = np.zeros((g0, g1), np.int32)
        jj = np.zeros((g0, g1), np.int32)
        for s0 in range(g0):
            r0, r1 = s0, nt - 1 - s0
            tiles = [(r0, j) for j in range(r0, nt)]
            tiles += [(r1, j) for j in range(r1, nt)]
            for s1, (ti, tj) in enumerate(tiles):
                ii[s0, s1], jj[s0, s1] = ti, tj
    else:
        tri = [(i, j) for i in range(nt) for j in range(i, nt)]
        g0, g1 = 1, len(tri)
        ii = np.asarray([t[0] for t in tri], np.int32).reshape(1, -1)
        jj = np.asarray([t[1] for t in tri], np.int32).reshape(1, -1)

    out = pl.pallas_call(
        _sym_tile_kernel,
        out_shape=jax.ShapeDtypeStruct((n_pad, 128), jnp.float32),
        grid_spec=pltpu.PrefetchScalarGridSpec(
            num_scalar_prefetch=2,
            grid=(g0, g1),
            in_specs=[
                pl.BlockSpec((T, D), lambda s0, s1, ii, jj: (ii[s0, s1], 0)),
                pl.BlockSpec((T, D), lambda s0, s1, ii, jj: (jj[s0, s1], 0)),
                pl.BlockSpec((1, 1, T), lambda s0, s1, ii, jj: (jj[s0, s1], 0, 0)),
            ],
            out_specs=pl.BlockSpec((T, 128), lambda s0, s1, ii, jj: (ii[s0, s1], 0)),
        ),
        compiler_params=pltpu.CompilerParams(
            dimension_semantics=("arbitrary", "arbitrary"),
            vmem_limit_bytes=100 * 1024 * 1024,
        ),
    )(jnp.asarray(ii), jnp.asarray(jj), data, data, qn)
    return out[:, 0]


def _cross_rowsums(a_data, b_data, qn_b, TM, TN):
    n_pad, D = a_data.shape
    m_pad, _ = b_data.shape
    out = pl.pallas_call(
        _cross_tile_kernel,
        out_shape=jax.ShapeDtypeStruct((n_pad, 128), jnp.float32),
        grid=(n_pad // TM, m_pad // TN),
        in_specs=[
            pl.BlockSpec((TM, D), lambda i, j: (i, 0)),
            pl.BlockSpec((TN, D), lambda i, j: (j, 0)),
            pl.BlockSpec((1, 1, TN), lambda i, j: (j, 0, 0)),
        ],
        out_specs=pl.BlockSpec((TM, 128), lambda i, j: (i, 0)),
        compiler_params=pltpu.CompilerParams(
            dimension_semantics=("arbitrary", "arbitrary"),
            vmem_limit_bytes=100 * 1024 * 1024,
        ),
    )(a_data, b_data, qn_b)
    return out[:, 0]


def _prep(P, T, ksize):
    n, d = P.shape
    P32 = P.astype(jnp.float32)
    q = _LOG2E / float(ksize)
    n_pad = _round_up(n, T)
    d_pad = _round_up(d, 128)
    if n_pad != n or d_pad != d:
        P32 = jnp.zeros((n_pad, d_pad), jnp.float32).at[:n, :d].set(P32)
    scaled = (P32 * math.sqrt(2.0 * q)).astype(jnp.float8_e4m3fn)
    qn = jnp.sum(P32 * P32, axis=-1) * q
    if n_pad != n:
        qn = jnp.where(jnp.arange(n_pad) < n, qn, _BIG)
    rowfac = jnp.exp2(-qn)
    return scaled, qn, rowfac


def kernel(X, Z):
    ksize = 64.0
    N, D = X.shape
    M, D2 = Z.shape
    assert D == D2, "feature dims must match"
    norm = math.sqrt(2.0 * math.pi * ksize)

    Tx = _pick_tile(N)
    Tz = _pick_tile(M)
    Xc, qn_x, fac_x = _prep(X, Tx, ksize)
    Zc, qn_z, fac_z = _prep(Z, Tz, ksize)
    ntx, ntz = Xc.shape[0] // Tx, Zc.shape[0] // Tz
    qx_rows = qn_x.reshape(ntx, 1, Tx)
    qz_rows = qn_z.reshape(ntz, 1, Tz)

    rs_xx = _sym_rowsums(Xc, qx_rows, Tx, ntx)
    rs_zz = _sym_rowsums(Zc, qz_rows, Tz, ntz)
    rs_xz = _cross_rowsums(Xc, Zc, qz_rows, Tx, Tz)

    s_xx = jnp.sum(fac_x * rs_xx)
    s_zz = jnp.sum(fac_z * rs_zz)
    s_xz = jnp.sum(fac_x * rs_xz)

    m_xx = s_xx / (norm * N * N)
    m_zz = s_zz / (norm * M * M)
    m_xz = s_xz / (norm * N * M)
    return jnp.log(jnp.sqrt(m_xx * m_zz + 1e-5) / (m_xz + 1e-5))
